# baseline (speedup 1.0000x reference)
import functools

import jax
import jax.numpy as jnp
from jax import lax
from jax.experimental import pallas as pl
from jax.experimental.pallas import tpu as pltpu
from jax.experimental.pallas import tpu_sc as plsc

B, T, V = 1024, 200, 1000
VP = 1024
NT = VP // 128
N = B * T
NSEG = 4
NROWS = N // NSEG
NW = 32
ROWS_PER_W = NROWS // NW
CHUNK = 32
NCHUNK = ROWS_PER_W // CHUNK


def _row_lse_body(table_ref, lse_ref):
    x = table_ref[...]
    m = jnp.max(x, axis=1)
    s = jnp.sum(jnp.exp(x - m[:, None]), axis=1)
    lse_ref[...] = jnp.log(s) + m


def _row_lse(table):
    return pl.pallas_call(
        _row_lse_body,
        out_shape=jax.ShapeDtypeStruct((V,), jnp.float32),
    )(table)


_sc_mesh = plsc.VectorSubcoreMesh(core_axis_name="c", subcore_axis_name="s")


@functools.partial(
    pl.kernel,
    mesh=_sc_mesh,
    out_type=(
        jax.ShapeDtypeStruct((NROWS, V), jnp.float32),
        jax.ShapeDtypeStruct((NW, 16), jnp.float32),
    ),
    scratch_types=[
        pltpu.VMEM((ROWS_PER_W,), jnp.int32),
        pltpu.VMEM((ROWS_PER_W,), jnp.int32),
        [pltpu.VMEM((CHUNK,), jnp.int32)] * 2,
        [pltpu.VMEM((CHUNK,), jnp.int32)] * 2,
        [pltpu.VMEM((CHUNK,), jnp.float32)] * 2,
        [pltpu.VMEM((CHUNK,), jnp.float32)] * 2,
        [pltpu.VMEM((CHUNK, V), jnp.float32)] * 2,
        [pltpu.VMEM((CHUNK, 128), jnp.float32)] * 2,
        pltpu.VMEM((16,), jnp.float32),
        [pltpu.SemaphoreType.DMA] * 2,
        [pltpu.SemaphoreType.DMA] * 2,
        [pltpu.SemaphoreType.DMA] * 2,
        [pltpu.SemaphoreType.DMA] * 2,
    ],
    compiler_params=pltpu.CompilerParams(use_tc_tiling_on_sc=True),
)
def _sc_gather(idx_hbm, tgt_hbm, lse_hbm, piece_hbm,
               tableflat_hbm, out_hbm, part_hbm,
               idx_all, tgt_all, idxb, flat, pk, ls, rows, tail,
               part_v, sg, swb, spk, sls):
    wid = lax.axis_index("s") * 2 + lax.axis_index("c")
    base = wid * ROWS_PER_W
    pltpu.sync_copy(idx_hbm.at[pl.ds(base, ROWS_PER_W)], idx_all)
    pltpu.sync_copy(tgt_hbm.at[pl.ds(base, ROWS_PER_W)], tgt_all)
    lanes = lax.iota(jnp.int32, 16)

    def issue(c, p):
        off = c * CHUNK
        for j in range(CHUNK // 16):
            idx16 = idx_all[pl.ds(off + j * 16, 16)]
            tgt16 = tgt_all[pl.ds(off + j * 16, 16)]
            idxb[p][pl.ds(j * 16, 16)] = idx16
            flat[p][pl.ds(j * 16, 16)] = idx16 * V + tgt16
        pltpu.async_copy(tableflat_hbm.at[flat[p]], pk[p], spk[p])
        pltpu.async_copy(lse_hbm.at[idxb[p]], ls[p], sls[p])
        for h in range(CHUNK // 16):
            idx16 = idx_all[pl.ds(off + h * 16, 16)]
            p8 = idx16 * NT
            for tc in range(NT - 1):
                pltpu.async_copy(
                    piece_hbm.at[p8 + tc],
                    rows[p].at[pl.ds(h * 16, 16), pl.ds(tc * 128, 128)],
                    sg[p])
            pltpu.async_copy(piece_hbm.at[p8 + (NT - 1)],
                             tail[p].at[pl.ds(h * 16, 16)],
                             sg[p])

    def wait_rows(p):
        for _ in range(2 * NT):
            pltpu.make_async_copy(
                piece_hbm.at[lanes],
                rows[p].at[pl.ds(0, 16), pl.ds(0, 128)],
                sg[p]).wait()

    def fix_tail(p):
        for r in range(CHUNK):
            rows[p][r, pl.ds(984, 16)] = tail[p][r, pl.ds(88, 16)]
            for j in range(6):
                rows[p][r, pl.ds(896 + 16 * j, 16)] = (
                    tail[p][r, pl.ds(16 * j, 16)])

    def start_wb(c, p):
        pltpu.async_copy(rows[p],
                         out_hbm.at[pl.ds(base + c * CHUNK, CHUNK)],
                         swb[p])

    def wait_wb(p):
        pltpu.make_async_copy(rows[p],
                              out_hbm.at[pl.ds(base, CHUNK)],
                              swb[p]).wait()

    def acc_chunk(p, acc):
        pltpu.make_async_copy(tableflat_hbm.at[flat[p]], pk[p],
                              spk[p]).wait()
        pltpu.make_async_copy(lse_hbm.at[idxb[p]], ls[p], sls[p]).wait()
        for j in range(CHUNK // 16):
            acc = acc + (ls[p][pl.ds(j * 16, 16)]
                         - pk[p][pl.ds(j * 16, 16)])
        return acc

    acc0 = jnp.zeros((16,), jnp.float32)
    issue(0, 0)
    issue(1, 1)
    wait_rows(0)
    fix_tail(0)
    start_wb(0, 0)
    acc0 = acc_chunk(0, acc0)

    def body(g, acc):
        c0 = 2 * g
        wait_wb(0)
        issue(c0, 0)
        wait_rows(1)
        fix_tail(1)
        start_wb(c0 - 1, 1)
        acc = acc_chunk(1, acc)
        wait_wb(1)
        issue(c0 + 1, 1)
        wait_rows(0)
        fix_tail(0)
        start_wb(c0, 0)
        acc = acc_chunk(0, acc)
        return acc

    acc0 = lax.fori_loop(1, NCHUNK // 2, body, acc0)
    wait_rows(1)
    fix_tail(1)
    start_wb(NCHUNK - 1, 1)
    acc0 = acc_chunk(1, acc0)
    wait_wb(0)
    wait_wb(1)
    part_v[...] = acc0
    pltpu.sync_copy(part_v, part_hbm.at[wid])


def kernel(index, targets, table):
    idx_flat = index.reshape(N)
    tgt_flat = targets.reshape(N)
    lse = _row_lse(table)
    pieces = jnp.pad(table, ((0, 0), (0, VP - V))).reshape(V * NT, 128)
    tableflat = lax.optimization_barrier(table).reshape(V * V)
    segs = []
    psum = jnp.zeros((), jnp.float32)
    for k in range(NSEG):
        sl = slice(k * NROWS, (k + 1) * NROWS)
        seg, partials = _sc_gather(idx_flat[sl], tgt_flat[sl], lse,
                                   pieces, tableflat)
        segs.append(seg)
        psum = psum + jnp.sum(partials)
    logits = jnp.concatenate(segs, axis=0)
    loss = psum * (1.0 / N)
    return (logits, loss)


# speedup vs baseline: 1.4615x; 1.4615x over previous
import functools

import jax
import jax.numpy as jnp
from jax import lax
from jax.experimental import pallas as pl
from jax.experimental.pallas import tpu as pltpu
from jax.experimental.pallas import tpu_sc as plsc

B, T, V = 1024, 200, 1000
VP = 1024
NT = VP // 128
N = B * T
NSEG = 4
NROWS = N // NSEG
NW = 32
ROWS_PER_W = NROWS // NW
CHUNK = 32
NCHUNK = ROWS_PER_W // CHUNK


def _row_lse_body(table_ref, lse_ref):
    x = table_ref[...]
    m = jnp.max(x, axis=1)
    s = jnp.sum(jnp.exp(x - m[:, None]), axis=1)
    lse_ref[...] = jnp.log(s) + m


def _row_lse(table):
    return pl.pallas_call(
        _row_lse_body,
        out_shape=jax.ShapeDtypeStruct((V,), jnp.float32),
    )(table)


TBLK = 512


def _xpose_body(seg_ref, out_ref):
    out_ref[...] = seg_ref[...].T


def _xpose_first(seg):
    return pl.pallas_call(
        _xpose_body,
        grid=(NROWS // TBLK,),
        in_specs=[pl.BlockSpec((TBLK, V), lambda i: (i, 0))],
        out_specs=pl.BlockSpec((V, TBLK), lambda i: (0, i)),
        out_shape=jax.ShapeDtypeStruct((V, N), jnp.float32),
    )(seg)


def _xpose_next_body(seg_ref, prev_ref, out_ref):
    out_ref[...] = seg_ref[...].T


def _xpose_next(prev, seg, k):
    off = k * (NROWS // TBLK)
    return pl.pallas_call(
        _xpose_next_body,
        grid=(NROWS // TBLK,),
        in_specs=[pl.BlockSpec((TBLK, V), lambda i: (i, 0)),
                  pl.BlockSpec(memory_space=pl.ANY)],
        out_specs=pl.BlockSpec((V, TBLK), lambda i, off=off: (0, off + i)),
        out_shape=jax.ShapeDtypeStruct((V, N), jnp.float32),
        input_output_aliases={1: 0},
    )(seg, prev)


_sc_mesh = plsc.VectorSubcoreMesh(core_axis_name="c", subcore_axis_name="s")


@functools.partial(
    pl.kernel,
    mesh=_sc_mesh,
    out_type=(
        jax.ShapeDtypeStruct((NROWS, V), jnp.float32),
        jax.ShapeDtypeStruct((NW, 16), jnp.float32),
    ),
    scratch_types=[
        pltpu.VMEM((ROWS_PER_W,), jnp.int32),
        pltpu.VMEM((ROWS_PER_W,), jnp.int32),
        [pltpu.VMEM((CHUNK,), jnp.int32)] * 2,
        [pltpu.VMEM((CHUNK,), jnp.int32)] * 2,
        [pltpu.VMEM((CHUNK,), jnp.float32)] * 2,
        [pltpu.VMEM((CHUNK,), jnp.float32)] * 2,
        [pltpu.VMEM((CHUNK, V), jnp.float32)] * 2,
        [pltpu.VMEM((CHUNK, 128), jnp.float32)] * 2,
        pltpu.VMEM((16,), jnp.float32),
        [pltpu.SemaphoreType.DMA] * 2,
        [pltpu.SemaphoreType.DMA] * 2,
        [pltpu.SemaphoreType.DMA] * 2,
        [pltpu.SemaphoreType.DMA] * 2,
    ],
    compiler_params=pltpu.CompilerParams(use_tc_tiling_on_sc=True),
)
def _sc_gather(idx_hbm, tgt_hbm, lse_hbm, piece_hbm,
               tableflat_hbm, out_hbm, part_hbm,
               idx_all, tgt_all, idxb, flat, pk, ls, rows, tail,
               part_v, sg, swb, spk, sls):
    wid = lax.axis_index("s") * 2 + lax.axis_index("c")
    base = wid * ROWS_PER_W
    pltpu.sync_copy(idx_hbm.at[pl.ds(base, ROWS_PER_W)], idx_all)
    pltpu.sync_copy(tgt_hbm.at[pl.ds(base, ROWS_PER_W)], tgt_all)
    lanes = lax.iota(jnp.int32, 16)

    def issue(c, p):
        off = c * CHUNK
        for j in range(CHUNK // 16):
            idx16 = idx_all[pl.ds(off + j * 16, 16)]
            tgt16 = tgt_all[pl.ds(off + j * 16, 16)]
            idxb[p][pl.ds(j * 16, 16)] = idx16
            flat[p][pl.ds(j * 16, 16)] = idx16 * V + tgt16
        pltpu.async_copy(tableflat_hbm.at[flat[p]], pk[p], spk[p])
        pltpu.async_copy(lse_hbm.at[idxb[p]], ls[p], sls[p])
        for h in range(CHUNK // 16):
            idx16 = idx_all[pl.ds(off + h * 16, 16)]
            p8 = idx16 * NT
            for tc in range(NT - 1):
                pltpu.async_copy(
                    piece_hbm.at[p8 + tc],
                    rows[p].at[pl.ds(h * 16, 16), pl.ds(tc * 128, 128)],
                    sg[p])
            pltpu.async_copy(piece_hbm.at[p8 + (NT - 1)],
                             tail[p].at[pl.ds(h * 16, 16)],
                             sg[p])

    def wait_rows(p):
        for _ in range(2 * NT):
            pltpu.make_async_copy(
                piece_hbm.at[lanes],
                rows[p].at[pl.ds(0, 16), pl.ds(0, 128)],
                sg[p]).wait()

    def fix_tail(p):
        for r in range(CHUNK):
            rows[p][r, pl.ds(984, 16)] = tail[p][r, pl.ds(88, 16)]
            for j in range(6):
                rows[p][r, pl.ds(896 + 16 * j, 16)] = (
                    tail[p][r, pl.ds(16 * j, 16)])

    def start_wb(c, p):
        pltpu.async_copy(rows[p],
                         out_hbm.at[pl.ds(base + c * CHUNK, CHUNK)],
                         swb[p])

    def wait_wb(p):
        pltpu.make_async_copy(rows[p],
                              out_hbm.at[pl.ds(base, CHUNK)],
                              swb[p]).wait()

    def acc_chunk(p, acc):
        pltpu.make_async_copy(tableflat_hbm.at[flat[p]], pk[p],
                              spk[p]).wait()
        pltpu.make_async_copy(lse_hbm.at[idxb[p]], ls[p], sls[p]).wait()
        for j in range(CHUNK // 16):
            acc = acc + (ls[p][pl.ds(j * 16, 16)]
                         - pk[p][pl.ds(j * 16, 16)])
        return acc

    acc0 = jnp.zeros((16,), jnp.float32)
    issue(0, 0)
    issue(1, 1)
    wait_rows(0)
    fix_tail(0)
    start_wb(0, 0)
    acc0 = acc_chunk(0, acc0)

    def body(g, acc):
        c0 = 2 * g
        wait_wb(0)
        issue(c0, 0)
        wait_rows(1)
        fix_tail(1)
        start_wb(c0 - 1, 1)
        acc = acc_chunk(1, acc)
        wait_wb(1)
        issue(c0 + 1, 1)
        wait_rows(0)
        fix_tail(0)
        start_wb(c0, 0)
        acc = acc_chunk(0, acc)
        return acc

    acc0 = lax.fori_loop(1, NCHUNK // 2, body, acc0)
    wait_rows(1)
    fix_tail(1)
    start_wb(NCHUNK - 1, 1)
    acc0 = acc_chunk(1, acc0)
    wait_wb(0)
    wait_wb(1)
    part_v[...] = acc0
    pltpu.sync_copy(part_v, part_hbm.at[wid])


def kernel(index, targets, table):
    idx_flat = index.reshape(N)
    tgt_flat = targets.reshape(N)
    lse = _row_lse(table)
    pieces = jnp.pad(table, ((0, 0), (0, VP - V))).reshape(V * NT, 128)
    tableflat = lax.optimization_barrier(table).reshape(V * V)
    psum = jnp.zeros((), jnp.float32)
    outT = None
    for k in range(NSEG):
        sl = slice(k * NROWS, (k + 1) * NROWS)
        seg, partials = _sc_gather(idx_flat[sl], tgt_flat[sl], lse,
                                   pieces, tableflat)
        outT = _xpose_first(seg) if k == 0 else _xpose_next(outT, seg, k)
        psum = psum + jnp.sum(partials)
    logits = outT.T
    loss = psum * (1.0 / N)
    return (logits, loss)


# speedup vs baseline: 1.4631x; 1.0011x over previous
import functools

import jax
import jax.numpy as jnp
from jax import lax
from jax.experimental import pallas as pl
from jax.experimental.pallas import tpu as pltpu
from jax.experimental.pallas import tpu_sc as plsc

B, T, V = 1024, 200, 1000
VP = 1024
NT = VP // 128
N = B * T
NSEG = 4
NROWS = N // NSEG
NW = 32
ROWS_PER_W = NROWS // NW
CHUNK = 32
NCHUNK = ROWS_PER_W // CHUNK


def _row_lse_body(table_ref, lse_ref):
    x = table_ref[...]
    m = jnp.max(x, axis=1)
    s = jnp.sum(jnp.exp(x - m[:, None]), axis=1)
    lse_ref[...] = jnp.log(s) + m


def _row_lse(table):
    return pl.pallas_call(
        _row_lse_body,
        out_shape=jax.ShapeDtypeStruct((V,), jnp.float32),
    )(table)


TBLK = 512


def _xpose_body(seg_ref, out_ref):
    out_ref[...] = seg_ref[...].T


def _xpose_first(seg):
    return pl.pallas_call(
        _xpose_body,
        grid=(NROWS // TBLK,),
        in_specs=[pl.BlockSpec((TBLK, V), lambda i: (i, 0))],
        out_specs=pl.BlockSpec((V, TBLK), lambda i: (0, i)),
        out_shape=jax.ShapeDtypeStruct((V, N), jnp.float32),
    )(seg)


def _xpose_next_body(seg_ref, prev_ref, out_ref):
    out_ref[...] = seg_ref[...].T


def _xpose_next(prev, seg, k):
    off = k * (NROWS // TBLK)
    return pl.pallas_call(
        _xpose_next_body,
        grid=(NROWS // TBLK,),
        in_specs=[pl.BlockSpec((TBLK, V), lambda i: (i, 0)),
                  pl.BlockSpec(memory_space=pl.ANY)],
        out_specs=pl.BlockSpec((V, TBLK), lambda i, off=off: (0, off + i)),
        out_shape=jax.ShapeDtypeStruct((V, N), jnp.float32),
        input_output_aliases={1: 0},
    )(seg, prev)


_sc_mesh = plsc.VectorSubcoreMesh(core_axis_name="c", subcore_axis_name="s")


@functools.partial(
    pl.kernel,
    mesh=_sc_mesh,
    out_type=(
        jax.ShapeDtypeStruct((NROWS, V), jnp.float32),
        jax.ShapeDtypeStruct((NW, 16), jnp.float32),
    ),
    scratch_types=[
        pltpu.VMEM((ROWS_PER_W,), jnp.int32),
        pltpu.VMEM((ROWS_PER_W,), jnp.int32),
        [pltpu.VMEM((CHUNK,), jnp.int32)] * 2,
        [pltpu.VMEM((CHUNK,), jnp.int32)] * 2,
        [pltpu.VMEM((CHUNK,), jnp.float32)] * 2,
        [pltpu.VMEM((CHUNK,), jnp.float32)] * 2,
        [pltpu.VMEM((CHUNK, V), jnp.float32)] * 2,
        [pltpu.VMEM((CHUNK, 128), jnp.float32)] * 2,
        pltpu.VMEM((16,), jnp.float32),
        [pltpu.SemaphoreType.DMA] * 2,
        [pltpu.SemaphoreType.DMA] * 2,
        [pltpu.SemaphoreType.DMA] * 2,
        [pltpu.SemaphoreType.DMA] * 2,
    ],
    compiler_params=pltpu.CompilerParams(use_tc_tiling_on_sc=True),
)
def _sc_gather(idx_hbm, tgt_hbm, lse_hbm, piece_hbm,
               tableflat_hbm, out_hbm, part_hbm,
               idx_all, tgt_all, idxb, flat, pk, ls, rows, tail,
               part_v, sg, swb, spk, sls):
    wid = lax.axis_index("s") * 2 + lax.axis_index("c")
    base = wid * ROWS_PER_W
    pltpu.sync_copy(idx_hbm.at[pl.ds(base, ROWS_PER_W)], idx_all)
    pltpu.sync_copy(tgt_hbm.at[pl.ds(base, ROWS_PER_W)], tgt_all)
    lanes = lax.iota(jnp.int32, 16)

    def issue(c, p):
        off = c * CHUNK
        for j in range(CHUNK // 16):
            idx16 = idx_all[pl.ds(off + j * 16, 16)]
            tgt16 = tgt_all[pl.ds(off + j * 16, 16)]
            idxb[p][pl.ds(j * 16, 16)] = idx16
            flat[p][pl.ds(j * 16, 16)] = idx16 * V + tgt16
        pltpu.async_copy(tableflat_hbm.at[flat[p]], pk[p], spk[p])
        pltpu.async_copy(lse_hbm.at[idxb[p]], ls[p], sls[p])
        for h in range(CHUNK // 16):
            idx16 = idx_all[pl.ds(off + h * 16, 16)]
            p8 = idx16 * NT
            for tc in range(NT - 1):
                pltpu.async_copy(
                    piece_hbm.at[p8 + tc],
                    rows[p].at[pl.ds(h * 16, 16), pl.ds(tc * 128, 128)],
                    sg[p])
            pltpu.async_copy(piece_hbm.at[p8 + (NT - 1)],
                             tail[p].at[pl.ds(h * 16, 16)],
                             sg[p])

    def wait_rows(p):
        for _ in range(2 * NT):
            pltpu.make_async_copy(
                piece_hbm.at[lanes],
                rows[p].at[pl.ds(0, 16), pl.ds(0, 128)],
                sg[p]).wait()

    def fix_tail(p):
        for r in range(CHUNK):
            rows[p][r, pl.ds(984, 16)] = tail[p][r, pl.ds(88, 16)]
            for j in range(6):
                rows[p][r, pl.ds(896 + 16 * j, 16)] = (
                    tail[p][r, pl.ds(16 * j, 16)])

    def start_wb(c, p):
        pltpu.async_copy(rows[p],
                         out_hbm.at[pl.ds(base + c * CHUNK, CHUNK)],
                         swb[p])

    def wait_wb(p):
        pltpu.make_async_copy(rows[p],
                              out_hbm.at[pl.ds(base, CHUNK)],
                              swb[p]).wait()

    def acc_chunk(p, acc):
        pltpu.make_async_copy(tableflat_hbm.at[flat[p]], pk[p],
                              spk[p]).wait()
        pltpu.make_async_copy(lse_hbm.at[idxb[p]], ls[p], sls[p]).wait()
        for j in range(CHUNK // 16):
            acc = acc + (ls[p][pl.ds(j * 16, 16)]
                         - pk[p][pl.ds(j * 16, 16)])
        return acc

    acc0 = jnp.zeros((16,), jnp.float32)
    issue(0, 0)
    issue(1, 1)
    wait_rows(0)
    fix_tail(0)
    start_wb(0, 0)
    acc0 = acc_chunk(0, acc0)

    def body(g, acc):
        c0 = 2 * g
        wait_wb(0)
        issue(c0, 0)
        wait_rows(1)
        fix_tail(1)
        start_wb(c0 - 1, 1)
        acc = acc_chunk(1, acc)
        wait_wb(1)
        issue(c0 + 1, 1)
        wait_rows(0)
        fix_tail(0)
        start_wb(c0, 0)
        acc = acc_chunk(0, acc)
        return acc

    acc0 = lax.fori_loop(1, NCHUNK // 2, body, acc0)
    wait_rows(1)
    fix_tail(1)
    start_wb(NCHUNK - 1, 1)
    acc0 = acc_chunk(1, acc0)
    wait_wb(0)
    wait_wb(1)
    part_v[...] = acc0
    pltpu.sync_copy(part_v, part_hbm.at[wid])


def kernel(index, targets, table):
    idx_flat = index.reshape(N)
    tgt_flat = targets.reshape(N)
    lse = _row_lse(table)
    pieces = jnp.pad(table, ((0, 0), (0, VP - V))).reshape(V * NT, 128)
    tableflat = lax.optimization_barrier(table).reshape(V * V)
    psum = jnp.zeros((), jnp.float32)
    outT = None
    dep = [jnp.float32(0)] * NSEG
    for k in range(NSEG):
        sl = slice(k * NROWS, (k + 1) * NROWS)
        lse_k = lse + dep[k - 2] if k >= 2 else lse
        seg, partials = _sc_gather(idx_flat[sl], tgt_flat[sl], lse_k,
                                   pieces, tableflat)
        outT = _xpose_first(seg) if k == 0 else _xpose_next(outT, seg, k)
        d = outT[0, 0]
        dep[k] = jnp.where(jnp.isnan(d), jnp.float32(0), jnp.float32(0))
        psum = psum + jnp.sum(partials)
    logits = outT.T
    loss = psum * (1.0 / N)
    return (logits, loss)


# speedup vs baseline: 1.4635x; 1.0003x over previous
import functools

import jax
import jax.numpy as jnp
from jax import lax
from jax.experimental import pallas as pl
from jax.experimental.pallas import tpu as pltpu
from jax.experimental.pallas import tpu_sc as plsc

B, T, V = 1024, 200, 1000
VP = 1024
NT = VP // 128
N = B * T
NSEG = 4
NROWS = N // NSEG
NW = 32
ROWS_PER_W = NROWS // NW
CHUNK = 32
NCHUNK = ROWS_PER_W // CHUNK


def _row_lse_body(table_ref, lse_ref):
    x = table_ref[...]
    m = jnp.max(x, axis=1)
    s = jnp.sum(jnp.exp(x - m[:, None]), axis=1)
    lse_ref[...] = jnp.log(s) + m


def _row_lse(table):
    return pl.pallas_call(
        _row_lse_body,
        out_shape=jax.ShapeDtypeStruct((V,), jnp.float32),
    )(table)


TBLK = 512


def _xpose_body(seg_ref, out_ref):
    out_ref[...] = seg_ref[...].T


def _xpose_first(seg):
    return pl.pallas_call(
        _xpose_body,
        grid=(NROWS // TBLK,),
        in_specs=[pl.BlockSpec((TBLK, V), lambda i: (i, 0))],
        out_specs=pl.BlockSpec((V, TBLK), lambda i: (0, i)),
        out_shape=jax.ShapeDtypeStruct((V, N), jnp.float32),
    )(seg)


def _xpose_next_body(seg_ref, prev_ref, out_ref):
    out_ref[...] = seg_ref[...].T


def _xpose_next(prev, seg, k):
    off = k * (NROWS // TBLK)
    return pl.pallas_call(
        _xpose_next_body,
        grid=(NROWS // TBLK,),
        in_specs=[pl.BlockSpec((TBLK, V), lambda i: (i, 0)),
                  pl.BlockSpec(memory_space=pl.ANY)],
        out_specs=pl.BlockSpec((V, TBLK), lambda i, off=off: (0, off + i)),
        out_shape=jax.ShapeDtypeStruct((V, N), jnp.float32),
        input_output_aliases={1: 0},
    )(seg, prev)


_sc_mesh = plsc.VectorSubcoreMesh(core_axis_name="c", subcore_axis_name="s")


@functools.partial(
    pl.kernel,
    mesh=_sc_mesh,
    out_type=(
        jax.ShapeDtypeStruct((NROWS, V), jnp.float32),
        jax.ShapeDtypeStruct((NW, 16), jnp.float32),
    ),
    scratch_types=[
        pltpu.VMEM((ROWS_PER_W,), jnp.int32),
        pltpu.VMEM((ROWS_PER_W,), jnp.int32),
        [pltpu.VMEM((CHUNK,), jnp.int32)] * 2,
        [pltpu.VMEM((CHUNK,), jnp.int32)] * 2,
        [pltpu.VMEM((CHUNK,), jnp.float32)] * 2,
        [pltpu.VMEM((CHUNK,), jnp.float32)] * 2,
        [pltpu.VMEM((CHUNK, V), jnp.float32)] * 2,
        [pltpu.VMEM((CHUNK, 128), jnp.float32)] * 2,
        pltpu.VMEM((16,), jnp.float32),
        [pltpu.SemaphoreType.DMA] * 2,
        [pltpu.SemaphoreType.DMA] * 2,
        [pltpu.SemaphoreType.DMA] * 2,
        [pltpu.SemaphoreType.DMA] * 2,
    ],
    compiler_params=pltpu.CompilerParams(use_tc_tiling_on_sc=True),
)
def _sc_gather(idx_hbm, tgt_hbm, lse_hbm, piece_hbm,
               tableflat_hbm, out_hbm, part_hbm,
               idx_all, tgt_all, idxb, flat, pk, ls, rows, tail,
               part_v, sg, swb, spk, sls):
    wid = lax.axis_index("s") * 2 + lax.axis_index("c")
    base = wid * ROWS_PER_W
    pltpu.sync_copy(idx_hbm.at[pl.ds(base, ROWS_PER_W)], idx_all)
    pltpu.sync_copy(tgt_hbm.at[pl.ds(base, ROWS_PER_W)], tgt_all)
    lanes = lax.iota(jnp.int32, 16)

    def issue(c, p):
        off = c * CHUNK
        for j in range(CHUNK // 16):
            idx16 = idx_all[pl.ds(off + j * 16, 16)]
            tgt16 = tgt_all[pl.ds(off + j * 16, 16)]
            idxb[p][pl.ds(j * 16, 16)] = idx16
            flat[p][pl.ds(j * 16, 16)] = idx16 * V + tgt16
        pltpu.async_copy(tableflat_hbm.at[flat[p]], pk[p], spk[p])
        pltpu.async_copy(lse_hbm.at[idxb[p]], ls[p], sls[p])
        for h in range(CHUNK // 16):
            idx16 = idx_all[pl.ds(off + h * 16, 16)]
            p8 = idx16 * NT
            for tc in range(NT - 1):
                pltpu.async_copy(
                    piece_hbm.at[p8 + tc],
                    rows[p].at[pl.ds(h * 16, 16), pl.ds(tc * 128, 128)],
                    sg[p])
            pltpu.async_copy(piece_hbm.at[p8 + (NT - 1)],
                             tail[p].at[pl.ds(h * 16, 16)],
                             sg[p])

    def wait_rows(p):
        for _ in range(2 * NT):
            pltpu.make_async_copy(
                piece_hbm.at[lanes],
                rows[p].at[pl.ds(0, 16), pl.ds(0, 128)],
                sg[p]).wait()

    def fix_tail(p):
        for r in range(CHUNK):
            rows[p][r, pl.ds(984, 16)] = tail[p][r, pl.ds(88, 16)]
            for j in range(6):
                rows[p][r, pl.ds(896 + 16 * j, 16)] = (
                    tail[p][r, pl.ds(16 * j, 16)])

    def start_wb(c, p):
        pltpu.async_copy(rows[p],
                         out_hbm.at[pl.ds(base + c * CHUNK, CHUNK)],
                         swb[p])

    def wait_wb(p):
        pltpu.make_async_copy(rows[p],
                              out_hbm.at[pl.ds(base, CHUNK)],
                              swb[p]).wait()

    def acc_chunk(p, acc):
        pltpu.make_async_copy(tableflat_hbm.at[flat[p]], pk[p],
                              spk[p]).wait()
        pltpu.make_async_copy(lse_hbm.at[idxb[p]], ls[p], sls[p]).wait()
        for j in range(CHUNK // 16):
            acc = acc + (ls[p][pl.ds(j * 16, 16)]
                         - pk[p][pl.ds(j * 16, 16)])
        return acc

    acc0 = jnp.zeros((16,), jnp.float32)
    issue(0, 0)
    issue(1, 1)
    wait_rows(0)
    fix_tail(0)
    start_wb(0, 0)
    acc0 = acc_chunk(0, acc0)

    def body(g, acc):
        c0 = 2 * g
        wait_wb(0)
        issue(c0, 0)
        wait_rows(1)
        fix_tail(1)
        start_wb(c0 - 1, 1)
        acc = acc_chunk(1, acc)
        wait_wb(1)
        issue(c0 + 1, 1)
        wait_rows(0)
        fix_tail(0)
        start_wb(c0, 0)
        acc = acc_chunk(0, acc)
        return acc

    acc0 = lax.fori_loop(1, NCHUNK // 2, body, acc0)
    wait_rows(1)
    fix_tail(1)
    start_wb(NCHUNK - 1, 1)
    acc0 = acc_chunk(1, acc0)
    wait_wb(0)
    wait_wb(1)
    part_v[...] = acc0
    pltpu.sync_copy(part_v, part_hbm.at[wid])


def kernel(index, targets, table):
    idx_flat = index.reshape(N)
    tgt_flat = targets.reshape(N)
    lse = _row_lse(table)
    pieces = jnp.pad(table, ((0, 0), (0, VP - V))).reshape(V * NT, 128)
    tableflat = lax.optimization_barrier(table).reshape(V * V)
    psum = jnp.zeros((), jnp.float32)
    outT = None
    for k in range(NSEG):
        sl = slice(k * NROWS, (k + 1) * NROWS)
        seg, partials = _sc_gather(idx_flat[sl], tgt_flat[sl], lse,
                                   pieces, tableflat)
        outT = _xpose_first(seg) if k == 0 else _xpose_next(outT, seg, k)
        psum = psum + jnp.sum(partials)
    logits = outT.T
    loss = psum * (1.0 / N)
    return (logits, loss)
